# baseline (speedup 1.0000x reference)
import functools

import jax
import jax.numpy as jnp
from jax import lax
from jax.experimental import pallas as pl
from jax.experimental.pallas import tpu as pltpu
from jax.experimental.pallas import tpu_sc as plsc

K = 32
TQ = 256
NPART = 4


def _stats_body(feat_ref, wf_ref, bnw_ref, bnb_ref, wfs_ref, bias_ref):
    f = feat_ref[...]
    mean = jnp.mean(f, axis=(0, 2))
    var = jnp.mean(jnp.square(f - mean[None, :, None]), axis=(0, 2))
    s = bnw_ref[...].reshape(-1) / jnp.sqrt(var + 1e-5)
    bp = bnb_ref[...].reshape(-1) - mean * s
    wf = wf_ref[...]
    wfs_ref[...] = wf * s[None, :]
    bias_ref[...] = lax.dot_general(
        wf, bp[:, None], (((1,), (0,)), ((), ())),
        preferred_element_type=jnp.float32).reshape(1, -1)


def _fold_bn(feat, w_f, bn0_w, bn0_b):
    CP, CH = w_f.shape
    return pl.pallas_call(
        _stats_body,
        out_shape=(
            jax.ShapeDtypeStruct((CP, CH), jnp.float32),
            jax.ShapeDtypeStruct((1, CP), jnp.float32),
        ),
    )(feat, w_f, bn0_w.reshape(1, CH), bn0_b.reshape(1, CH))


def _feat0_body(xq_ref, fb_ref, wfs_ref, wx_ref, bias_ref, f0r_ref):
    f0r = lax.dot_general(fb_ref[0], wfs_ref[...], (((0,), (1,)), ((), ())),
                          preferred_element_type=jnp.float32)
    f0r = f0r + lax.dot_general(xq_ref[0], wx_ref[...],
                                (((1,), (1,)), ((), ())),
                                preferred_element_type=jnp.float32)
    f0r_ref[0] = f0r + bias_ref[...]


def _feat0(xyz, feat, wfs, wx, bias):
    B, N, _ = xyz.shape
    CH = feat.shape[1]
    CP = wfs.shape[0]
    return pl.pallas_call(
        _feat0_body,
        grid=(B, N // TQ),
        in_specs=[
            pl.BlockSpec((1, TQ, 3), lambda b, i: (b, i, 0)),
            pl.BlockSpec((1, CH, TQ), lambda b, i: (b, 0, i)),
            pl.BlockSpec((CP, CH), lambda b, i: (0, 0)),
            pl.BlockSpec((CP, 3), lambda b, i: (0, 0)),
            pl.BlockSpec((1, CP), lambda b, i: (0, 0)),
        ],
        out_specs=pl.BlockSpec((1, TQ, CP), lambda b, i: (b, i, 0)),
        out_shape=jax.ShapeDtypeStruct((B, N, CP), jnp.float32),
    )(xyz, feat, wfs, wx, bias)


def _make_knn_body(b_const, n):
    def body(xq_ref, xa_ref, idx_ref):
        xq = xq_ref[0]
        xa = xa_ref[0]
        inner = lax.dot_general(xq, xa, (((1,), (1,)), ((), ())),
                                preferred_element_type=jnp.float32)
        xx = jnp.sum(xa * xa, axis=1)
        score = 2.0 * inner - xx[None, :]
        iota_f = lax.broadcasted_iota(
            jnp.int32, (TQ, n), 1).astype(jnp.float32)
        cols = []
        for _ in range(K):
            m = jnp.max(score, axis=1, keepdims=True)
            ge = score >= m
            j = jnp.min(jnp.where(ge, iota_f, float(n)), axis=1)
            cols.append(j)
            score = jnp.where(ge, -jnp.inf, score)
        idx = jnp.stack(cols, axis=1).astype(jnp.int32)
        idx_ref[0] = idx + b_const * n
    return body


def _knn_part(xyz, b_const, h_const):
    B, N, _ = xyz.shape
    np_ = N // 2
    steps = np_ // TQ
    return pl.pallas_call(
        _make_knn_body(b_const, N),
        grid=(steps,),
        in_specs=[
            pl.BlockSpec((1, TQ, 3),
                         lambda i: (b_const, h_const * steps + i, 0)),
            pl.BlockSpec((1, N, 3), lambda i: (b_const, 0, 0)),
        ],
        out_specs=pl.BlockSpec((1, TQ, K), lambda i: (0, i, 0)),
        out_shape=jax.ShapeDtypeStruct((1, np_, K), jnp.int32),
    )(xyz, xyz)


def _gather_max_sc(table, idx2d, n_q, cp):
    rows_per_chunk = 128
    q_per_chunk = rows_per_chunk // K
    n_workers = 32
    q_per_w = n_q // n_workers
    chunks = q_per_w * K // rows_per_chunk
    mesh = plsc.VectorSubcoreMesh(core_axis_name="c", subcore_axis_name="s")

    @functools.partial(
        pl.kernel,
        out_type=jax.ShapeDtypeStruct((n_q, cp), jnp.float32),
        mesh=mesh,
        scratch_types=[
            pltpu.VMEM((chunks, 128), jnp.int32),
            pltpu.VMEM((rows_per_chunk, cp), jnp.float32),
            pltpu.VMEM((rows_per_chunk, cp), jnp.float32),
            pltpu.VMEM((q_per_w, cp), jnp.float32),
            pltpu.SemaphoreType.DMA,
            pltpu.SemaphoreType.DMA,
        ],
    )
    def kb(table_hbm, idx_hbm, out_hbm, idx_v, rows0, rows1, out_v, s0, s1):
        wid = lax.axis_index("s") * 2 + lax.axis_index("c")
        pltpu.sync_copy(idx_hbm.at[pl.ds(wid * chunks, chunks)], idx_v)
        bufs = (rows0, rows1)
        sems = (s0, s1)
        pltpu.async_copy(table_hbm.at[idx_v.at[0]], rows0, s0)
        pltpu.async_copy(table_hbm.at[idx_v.at[1]], rows1, s1)

        def pair_body(p, _):
            for b in range(2):
                ch = p * 2 + b
                rows_v, sem = bufs[b], sems[b]
                pltpu.make_async_copy(table_hbm.at[idx_v.at[ch]], rows_v,
                                      sem).wait()

                def col_body(c, _):
                    off = pl.ds(pl.multiple_of(c * 16, 16), 16)
                    for q in range(q_per_chunk):
                        vals = [rows_v[q * K + j, off] for j in range(K)]
                        while len(vals) > 1:
                            vals = [jnp.maximum(vals[i], vals[i + 1])
                                    for i in range(0, len(vals) - 1, 2)] + (
                                        [vals[-1]] if len(vals) % 2 else [])
                        out_v[ch * q_per_chunk + q, off] = vals[0]
                    return 0

                lax.fori_loop(0, cp // 16, col_body, 0)

                @pl.when(ch + 2 < chunks)
                def _():
                    pltpu.async_copy(table_hbm.at[idx_v.at[ch + 2]], rows_v,
                                     sem)

            return 0

        lax.fori_loop(0, chunks // 2, pair_body, 0)
        pltpu.sync_copy(out_v, out_hbm.at[pl.ds(wid * q_per_w, q_per_w)])

    return kb(table, idx2d)


def _out_body(fr_ref, f0_ref, whs_ref, out_ref):
    diff = fr_ref[0] - f0_ref[0]
    out_ref[0] = lax.dot_general(whs_ref[...], diff, (((1,), (1,)), ((), ())),
                                 preferred_element_type=jnp.float32)


def _out_part(f_rows_p, f0_rows, whs, b_const, h_const):
    n_q, CP = f_rows_p.shape
    CH = whs.shape[0]
    steps = n_q // TQ
    return pl.pallas_call(
        _out_body,
        grid=(steps,),
        in_specs=[
            pl.BlockSpec((1, TQ, CP), lambda i: (0, i, 0)),
            pl.BlockSpec((1, TQ, CP),
                         lambda i: (b_const, h_const * steps + i, 0)),
            pl.BlockSpec((CH, CP), lambda i: (0, 0)),
        ],
        out_specs=pl.BlockSpec((1, CH, TQ), lambda i: (0, 0, i)),
        out_shape=jax.ShapeDtypeStruct((1, CH, n_q), jnp.float32),
    )(f_rows_p.reshape(1, n_q, CP), f0_rows, whs)


def kernel(xyz, feat, W_g, W_h, bn0_w, bn0_b):
    B, N, _ = xyz.shape
    CH = feat.shape[1]
    CP = W_g.shape[0]
    M = W_h.shape[1] // CP
    n_q = N // 2

    w_f = W_g[:, :CH]
    w_x = W_g[:, CH:]
    whs = W_h.reshape(CH, M, CP).sum(axis=1)

    wfs, bias = _fold_bn(feat, w_f, bn0_w, bn0_b)
    f0_rows = _feat0(xyz, feat, wfs, w_x, bias)
    table = f0_rows.reshape(B * N, CP)

    outs = []
    for b in range(B):
        for h in range(2):
            idx_p = _knn_part(xyz, b, h)
            idx2d = idx_p.reshape(n_q * K // 128, 128)
            f_rows_p = _gather_max_sc(table, idx2d, n_q, CP)
            outs.append(_out_part(f_rows_p, f0_rows, whs, b, h))

    return jnp.concatenate(
        [jnp.concatenate(outs[2 * b:2 * b + 2], axis=2) for b in range(B)],
        axis=0)
